# baseline (speedup 1.0000x reference)
import jax
import jax.numpy as jnp
from jax.experimental import pallas as pl
from jax.experimental.pallas import tpu as pltpu

_NCHUNK = 8


def _dma_body(kv_ref, vv_ref, kc_ref, vc_ref, ko_ref, vo_ref, sems):
    BH, L, D = kc_ref.shape
    S = kv_ref.shape[1]
    CH = BH // _NCHUNK
    copies = []
    i = 0
    for src, val, dst in ((kc_ref, kv_ref, ko_ref), (vc_ref, vv_ref, vo_ref)):
        for c in range(_NCHUNK):
            sl = pl.ds(c * CH, CH)
            copies.append(pltpu.make_async_copy(
                src.at[sl, pl.ds(S, L - S), :], dst.at[sl, pl.ds(S, L - S), :],
                sems.at[i]))
            i += 1
        copies.append(pltpu.make_async_copy(val, dst.at[:, pl.ds(0, S), :], sems.at[i]))
        i += 1
    for c in copies:
        c.start()
    for c in copies:
        c.wait()


def kernel(input_pos, k_val, v_val, k_cache, v_cache, pos):
    B, H, S_new, D = k_val.shape
    L = k_cache.shape[2]
    BH = B * H
    kc = k_cache.reshape(BH, L, D)
    vc = v_cache.reshape(BH, L, D)
    kv = k_val.reshape(BH, S_new, D)
    vv = v_val.reshape(BH, S_new, D)

    hbm = pl.BlockSpec(memory_space=pltpu.MemorySpace.HBM)
    nsem = 2 * (_NCHUNK + 1)
    ko, vo = pl.pallas_call(
        _dma_body,
        in_specs=[hbm, hbm, hbm, hbm],
        out_specs=[hbm, hbm],
        out_shape=[
            jax.ShapeDtypeStruct((BH, L, D), k_cache.dtype),
            jax.ShapeDtypeStruct((BH, L, D), v_cache.dtype),
        ],
        scratch_shapes=[pltpu.SemaphoreType.DMA((nsem,))],
    )(kv, vv, kc, vc)
    return (ko.reshape(B, H, L, D), vo.reshape(B, H, L, D))
